# baseline (speedup 1.0000x reference)
import functools

import jax
import jax.numpy as jnp
from jax import lax
from jax.experimental import pallas as pl
from jax.experimental.pallas import tpu as pltpu
from jax.experimental.pallas import tpu_sc as plsc

_BBLK = 128
_NBUF = 4


@functools.cache
def _build(b, l, d, num_cores, num_subcores):
    nw = num_cores * num_subcores
    assert b == nw * _BBLK and d == 64 and l % 2 == 0
    di = d // 8
    mesh = plsc.VectorSubcoreMesh(core_axis_name="c", subcore_axis_name="s")

    @functools.partial(
        pl.kernel,
        mesh=mesh,
        out_type=(
            jax.ShapeDtypeStruct((l, di, nw, 8, _BBLK), jnp.float32),
            jax.ShapeDtypeStruct((l, di, nw, 8, _BBLK), jnp.float32),
        ),
        scratch_types=[
            pltpu.VMEM((l, _BBLK), jnp.int32),
            pltpu.VMEM((l, _BBLK), jnp.int32),
        ] + [pltpu.VMEM((_BBLK, d), jnp.float32)] * _NBUF
          + [pltpu.VMEM((di, 8, _BBLK + 1), jnp.float32)] * _NBUF
          + [pltpu.SemaphoreType.DMA] * (2 * _NBUF),
        compiler_params=pltpu.CompilerParams(
            use_tc_tiling_on_sc=False, needs_layout_passes=False),
    )
    def k(src_idsT, tgt_idsT, src_tab, tgt_tab, src_out, tgt_out,
          sidx, tidx, *bufs_flat):
        rows_b = bufs_flat[:_NBUF]
        trans_b = bufs_flat[_NBUF:2 * _NBUF]
        sg_b = bufs_flat[2 * _NBUF:3 * _NBUF]
        sw_b = bufs_flat[3 * _NBUF:4 * _NBUF]
        wid = lax.axis_index("s") * num_cores + lax.axis_index("c")

        pltpu.sync_copy(src_idsT.at[pl.ds(0, l), pl.ds(wid * _BBLK, _BBLK)],
                        sidx)
        pltpu.sync_copy(tgt_idsT.at[pl.ds(0, l), pl.ds(wid * _BBLK, _BBLK)],
                        tidx)

        lanes = lax.iota(jnp.int32, 16)
        zeros16 = jnp.full((16,), 0, jnp.int32)
        ivecs = [(lanes + 16 * kk) // 8 for kk in range(d // 16)]
        rvecs = [(lanes + 16 * kk) % 8 for kk in range(d // 16)]

        def transpose(rows, trans):
            @plsc.parallel_loop(0, _BBLK, step=1, unroll=32)
            def tbody(c):
                cvec = zeros16 + c
                for kk in range(d // 16):
                    v = rows[c, pl.ds(16 * kk, 16)]
                    plsc.store_scatter(trans, [ivecs[kk], rvecs[kk], cvec], v)

        def do_table(tab, out, idx_v):
            def gather(s, slot):
                pltpu.async_copy(tab.at[idx_v.at[s]], rows_b[slot], sg_b[slot])

            def wait_g(slot):
                pltpu.make_async_copy(
                    tab.at[pl.ds(0, _BBLK)], rows_b[slot], sg_b[slot]).wait()

            def write(s, slot):
                trans = trans_b[slot]
                pltpu.async_copy(
                    trans.at[pl.ds(0, di), pl.ds(0, 8), pl.ds(0, _BBLK)],
                    out.at[s, pl.ds(0, di), wid], sw_b[slot])

            def wait_w(slot):
                trans = trans_b[slot]
                pltpu.make_async_copy(
                    trans.at[pl.ds(0, di), pl.ds(0, 8), pl.ds(0, _BBLK)],
                    out.at[0, pl.ds(0, di), 0], sw_b[slot]).wait()

            for slot in range(_NBUF):
                gather(slot, slot)

            n_t = l // _NBUF + 2

            def body(t, carry):
                for slot in range(_NBUF):
                    s = _NBUF * t + slot

                    @pl.when(s < l)
                    def _():
                        wait_g(slot)

                    @pl.when(jnp.logical_and(s >= _NBUF, s - _NBUF < l))
                    def _():
                        wait_w(slot)

                    @pl.when(s < l)
                    def _():
                        transpose(rows_b[slot], trans_b[slot])
                        write(s, slot)

                    @pl.when(s + _NBUF < l)
                    def _():
                        gather(s + _NBUF, slot)

                return carry

            lax.fori_loop(0, n_t, body, 0)

        do_table(src_tab, src_out, sidx)
        do_table(tgt_tab, tgt_out, tidx)

    return k


def kernel(src_ids, tgt_ids, src_table, tgt_table):
    b, l = src_ids.shape
    d = src_table.shape[1]
    info = plsc.get_sparse_core_info()
    nw = info.num_cores * info.num_subcores
    k = _build(b, l, d, info.num_cores, info.num_subcores)
    src_idsT = jnp.transpose(src_ids).astype(jnp.int32)
    tgt_idsT = jnp.transpose(tgt_ids).astype(jnp.int32)
    s5, t5 = k(src_idsT, tgt_idsT, src_table, tgt_table)

    def unshuffle(o):
        return o.transpose(2, 4, 0, 1, 3).reshape(b, l, d)

    return unshuffle(s5), unshuffle(t5)


# speedup vs baseline: 1.0729x; 1.0729x over previous
import functools

import jax
import jax.numpy as jnp
from jax import lax
from jax.experimental import pallas as pl
from jax.experimental.pallas import tpu as pltpu
from jax.experimental.pallas import tpu_sc as plsc

_BBLK = 128


@functools.cache
def _build(b, l, d, num_cores, num_subcores):
    nw = num_cores * num_subcores
    assert b == nw * _BBLK and d == 64 and l % 2 == 0
    di = d // 8
    mesh = plsc.VectorSubcoreMesh(core_axis_name="c", subcore_axis_name="s")

    @functools.partial(
        pl.kernel,
        mesh=mesh,
        out_type=(
            jax.ShapeDtypeStruct((l, di, nw, 8, _BBLK), jnp.float32),
            jax.ShapeDtypeStruct((l, di, nw, 8, _BBLK), jnp.float32),
        ),
        scratch_types=[
            pltpu.VMEM((l, _BBLK), jnp.int32),
            pltpu.VMEM((l, _BBLK), jnp.int32),
            pltpu.VMEM((_BBLK, d), jnp.float32),
            pltpu.VMEM((_BBLK, d), jnp.float32),
            pltpu.VMEM((di, 8, _BBLK + 1), jnp.float32),
            pltpu.VMEM((di, 8, _BBLK + 1), jnp.float32),
            pltpu.SemaphoreType.DMA,
            pltpu.SemaphoreType.DMA,
            pltpu.SemaphoreType.DMA,
            pltpu.SemaphoreType.DMA,
        ],
        compiler_params=pltpu.CompilerParams(
            use_tc_tiling_on_sc=False, needs_layout_passes=False),
    )
    def k(src_idsT, tgt_idsT, src_tab, tgt_tab, src_out, tgt_out,
          sidx, tidx, rows0, rows1, trans0, trans1, sg0, sg1, sw0, sw1):
        wid = lax.axis_index("s") * num_cores + lax.axis_index("c")

        pltpu.sync_copy(src_idsT.at[pl.ds(0, l), pl.ds(wid * _BBLK, _BBLK)],
                        sidx)
        pltpu.sync_copy(tgt_idsT.at[pl.ds(0, l), pl.ds(wid * _BBLK, _BBLK)],
                        tidx)

        lanes = lax.iota(jnp.int32, 16)
        zeros16 = jnp.full((16,), 0, jnp.int32)
        ivecs = [(lanes + 16 * kk) // 8 for kk in range(d // 16)]
        rvecs = [(lanes + 16 * kk) % 8 for kk in range(d // 16)]

        def transpose(rows, trans):
            @plsc.parallel_loop(0, _BBLK, step=1, unroll=32)
            def tbody(c):
                cvec = zeros16 + c
                for kk in range(d // 16):
                    v = rows[c, pl.ds(16 * kk, 16)]
                    plsc.store_scatter(trans, [ivecs[kk], rvecs[kk], cvec], v)

        def do_table(tab, out, idx_v):
            bufs = ((rows0, trans0, sg0, sw0), (rows1, trans1, sg1, sw1))

            def gather(s, slot):
                rows, _, sg, _ = bufs[slot]
                pltpu.async_copy(tab.at[idx_v.at[s]], rows, sg)

            def wait_g(slot):
                rows, _, sg, _ = bufs[slot]
                pltpu.make_async_copy(tab.at[pl.ds(0, _BBLK)], rows, sg).wait()

            def write(s, slot):
                _, trans, _, sw = bufs[slot]
                pltpu.async_copy(
                    trans.at[pl.ds(0, di), pl.ds(0, 8), pl.ds(0, _BBLK)],
                    out.at[s, pl.ds(0, di), wid], sw)

            def wait_w(slot):
                _, trans, _, sw = bufs[slot]
                pltpu.make_async_copy(
                    trans.at[pl.ds(0, di), pl.ds(0, 8), pl.ds(0, _BBLK)],
                    out.at[0, pl.ds(0, di), 0], sw).wait()

            gather(0, 0)
            gather(1, 1)

            def body(t, carry):
                for slot in (0, 1):
                    s = 2 * t + slot
                    rows, trans, _, _ = bufs[slot]
                    wait_g(slot)

                    @pl.when(t > 0)
                    def _():
                        wait_w(slot)

                    transpose(rows, trans)
                    write(s, slot)

                    @pl.when(s + 2 < l)
                    def _():
                        gather(s + 2, slot)

                return carry

            lax.fori_loop(0, l // 2, body, 0)
            wait_w(0)
            wait_w(1)

        do_table(src_tab, src_out, sidx)
        do_table(tgt_tab, tgt_out, tidx)

    return k


def kernel(src_ids, tgt_ids, src_table, tgt_table):
    b, l = src_ids.shape
    d = src_table.shape[1]
    info = plsc.get_sparse_core_info()
    nw = info.num_cores * info.num_subcores
    k = _build(b, l, d, info.num_cores, info.num_subcores)
    src_idsT = jnp.transpose(src_ids).astype(jnp.int32)
    tgt_idsT = jnp.transpose(tgt_ids).astype(jnp.int32)
    s5, t5 = k(src_idsT, tgt_idsT, src_table, tgt_table)

    def unshuffle(o):
        return o.transpose(2, 4, 0, 1, 3).reshape(b, l, d)

    return unshuffle(s5), unshuffle(t5)


# speedup vs baseline: 1.1004x; 1.0256x over previous
import functools

import jax
import jax.numpy as jnp
from jax import lax
from jax.experimental import pallas as pl
from jax.experimental.pallas import tpu as pltpu
from jax.experimental.pallas import tpu_sc as plsc

_BBLK = 128


@functools.cache
def _build(b, l, d, num_cores, num_subcores):
    nw = num_cores * num_subcores
    assert b == nw * _BBLK and d == 64 and l % 2 == 0
    di = d // 8
    mesh = plsc.VectorSubcoreMesh(core_axis_name="c", subcore_axis_name="s")

    @functools.partial(
        pl.kernel,
        mesh=mesh,
        out_type=(
            jax.ShapeDtypeStruct((l, di, nw, 8, _BBLK), jnp.float32),
            jax.ShapeDtypeStruct((l, di, nw, 8, _BBLK), jnp.float32),
        ),
        scratch_types=[
            pltpu.VMEM((l, _BBLK), jnp.int32),
            pltpu.VMEM((l, _BBLK), jnp.int32),
            pltpu.VMEM((_BBLK, d), jnp.float32),
            pltpu.VMEM((_BBLK, d), jnp.float32),
            pltpu.VMEM((di, 8, _BBLK + 1), jnp.float32),
            pltpu.VMEM((di, 8, _BBLK + 1), jnp.float32),
            pltpu.SemaphoreType.DMA,
            pltpu.SemaphoreType.DMA,
            pltpu.SemaphoreType.DMA,
            pltpu.SemaphoreType.DMA,
        ],
        compiler_params=pltpu.CompilerParams(
            use_tc_tiling_on_sc=False, needs_layout_passes=False),
    )
    def k(src_idsT, tgt_idsT, src_tab, tgt_tab, src_out, tgt_out,
          sidx, tidx, rows0, rows1, trans0, trans1, sg0, sg1, sw0, sw1):
        wid = lax.axis_index("s") * num_cores + lax.axis_index("c")

        pltpu.sync_copy(src_idsT.at[pl.ds(0, l), pl.ds(wid * _BBLK, _BBLK)],
                        sidx)
        pltpu.sync_copy(tgt_idsT.at[pl.ds(0, l), pl.ds(wid * _BBLK, _BBLK)],
                        tidx)

        lanes = lax.iota(jnp.int32, 16)
        zeros16 = jnp.full((16,), 0, jnp.int32)
        ivecs = [(lanes + 16 * kk) // 8 for kk in range(d // 16)]
        rvecs = [(lanes + 16 * kk) % 8 for kk in range(d // 16)]

        def transpose(rows, trans):
            @plsc.parallel_loop(0, _BBLK, step=1, unroll=16)
            def tbody(c):
                cvec = zeros16 + c
                for kk in range(d // 16):
                    v = rows[c, pl.ds(16 * kk, 16)]
                    plsc.store_scatter(trans, [ivecs[kk], rvecs[kk], cvec], v)

        def do_table(tab, out, idx_v):
            bufs = ((rows0, trans0, sg0, sw0), (rows1, trans1, sg1, sw1))

            def gather(s, slot):
                rows, _, sg, _ = bufs[slot]
                pltpu.async_copy(tab.at[idx_v.at[s]], rows, sg)

            def wait_g(slot):
                rows, _, sg, _ = bufs[slot]
                pltpu.make_async_copy(tab.at[pl.ds(0, _BBLK)], rows, sg).wait()

            def write(s, slot):
                _, trans, _, sw = bufs[slot]
                pltpu.async_copy(
                    trans.at[pl.ds(0, di), pl.ds(0, 8), pl.ds(0, _BBLK)],
                    out.at[s, pl.ds(0, di), wid], sw)

            def wait_w(slot):
                _, trans, _, sw = bufs[slot]
                pltpu.make_async_copy(
                    trans.at[pl.ds(0, di), pl.ds(0, 8), pl.ds(0, _BBLK)],
                    out.at[0, pl.ds(0, di), 0], sw).wait()

            gather(0, 0)
            gather(1, 1)

            def body(t, carry):
                for slot in (0, 1):
                    s = 2 * t + slot
                    rows, trans, _, _ = bufs[slot]
                    wait_g(slot)

                    @pl.when(t > 0)
                    def _():
                        wait_w(slot)

                    transpose(rows, trans)
                    write(s, slot)

                    @pl.when(s + 2 < l)
                    def _():
                        gather(s + 2, slot)

                return carry

            lax.fori_loop(0, l // 2, body, 0)
            wait_w(0)
            wait_w(1)

        do_table(src_tab, src_out, sidx)
        do_table(tgt_tab, tgt_out, tidx)

    return k


def kernel(src_ids, tgt_ids, src_table, tgt_table):
    b, l = src_ids.shape
    d = src_table.shape[1]
    info = plsc.get_sparse_core_info()
    nw = info.num_cores * info.num_subcores
    k = _build(b, l, d, info.num_cores, info.num_subcores)
    src_idsT = jnp.transpose(src_ids).astype(jnp.int32)
    tgt_idsT = jnp.transpose(tgt_ids).astype(jnp.int32)
    s5, t5 = k(src_idsT, tgt_idsT, src_table, tgt_table)

    def unshuffle(o):
        return o.transpose(2, 4, 0, 1, 3).reshape(b, l, d)

    return unshuffle(s5), unshuffle(t5)
